# baseline (speedup 1.0000x reference)
import functools

import jax
import jax.numpy as jnp
from jax import lax
from jax.experimental import pallas as pl
from jax.experimental.pallas import tpu as pltpu
from jax.experimental.pallas import tpu_sc as plsc

N_NODES = 10000
N_EDGES = 320000

_NCORES = 2
_NSUB = 16
_CHUNK = 80


def _make_segsum(n, e, d, nbuf, tc_tiling):
    nw = _NCORES * _NSUB
    epw = e // nw
    nch = epw // _CHUNK
    rps = (n // _NSUB) // 8 * 8
    tail = n - rps * _NSUB
    assert tail % 8 == 0

    mesh = plsc.VectorSubcoreMesh(core_axis_name="c", subcore_axis_name="s")

    @functools.partial(
        pl.kernel,
        out_type=jax.ShapeDtypeStruct((_NCORES * n, d), jnp.float32),
        mesh=mesh,
        compiler_params=pltpu.CompilerParams(use_tc_tiling_on_sc=tc_tiling),
        scratch_types=[
            pltpu.VMEM((epw,), jnp.int32),
            pltpu.VMEM((epw,), jnp.int32),
            pltpu.VMEM((nbuf, _CHUNK, d), jnp.float32),
            pltpu.VMEM_SHARED((n, d), jnp.float32),
            pltpu.SemaphoreType.DMA((nbuf,)),
        ],
    )
    def segsum(x_hbm, edges_hbm, zeros_hbm, out_hbm,
               srcbuf, dstbuf, rows, acc, sems):
        c = lax.axis_index("c")
        s = lax.axis_index("s")
        w = c * _NSUB + s
        pltpu.sync_copy(zeros_hbm.at[pl.ds(0, rps)],
                        acc.at[pl.ds(s * rps, rps)])

        @pl.when(s == 0)
        def _():
            pltpu.sync_copy(zeros_hbm.at[pl.ds(0, tail)],
                            acc.at[pl.ds(rps * _NSUB, tail)])

        base = pl.multiple_of(w * epw, 8)
        pltpu.sync_copy(edges_hbm.at[0, pl.ds(base, epw)], srcbuf)
        pltpu.sync_copy(edges_hbm.at[1, pl.ds(base, epw)], dstbuf)
        plsc.subcore_barrier()

        def src_idx(g):
            return srcbuf.at[pl.ds(pl.multiple_of(g * _CHUNK, 8), _CHUNK)]

        def dst_idx(g):
            return dstbuf.at[pl.ds(pl.multiple_of(g * _CHUNK, 8), _CHUNK)]

        for b in range(nbuf):
            pltpu.async_copy(x_hbm.at[src_idx(b)], rows.at[b], sems.at[b])

        @pl.loop(0, nch, step=nbuf)
        def _(g0):
            for b in range(nbuf):
                g = g0 + b

                @pl.when(g < nch)
                def _():
                    pltpu.make_async_copy(x_hbm.at[src_idx(g)], rows.at[b],
                                          sems.at[b]).wait()
                    pltpu.sync_copy(rows.at[b], acc.at[dst_idx(g)],
                                    add=True)
                    nxt = g + nbuf

                    @pl.when(nxt < nch)
                    def _():
                        pltpu.async_copy(x_hbm.at[src_idx(nxt)],
                                         rows.at[b], sems.at[b])

        plsc.subcore_barrier()
        pltpu.sync_copy(acc.at[pl.ds(s * rps, rps)],
                        out_hbm.at[pl.ds(c * n + s * rps, rps)])

        @pl.when(s == 0)
        def _():
            pltpu.sync_copy(acc.at[pl.ds(rps * _NSUB, tail)],
                            out_hbm.at[pl.ds(c * n + rps * _NSUB, tail)])

    return segsum


_segsum64 = _make_segsum(N_NODES, N_EDGES, 64, 10, False)

_BLOCK = 2000
_NBLK = N_NODES // _BLOCK


def _row_spec(d):
    return pl.BlockSpec((_BLOCK, d), lambda i: (i, 0))


def _half_spec(d, half):
    return pl.BlockSpec((_BLOCK, d), lambda i, h=half: (i + h * _NBLK, 0))


def _full_spec(r, c):
    return pl.BlockSpec((r, c), lambda i: (0, 0))


def _proj_body(x_ref, w_ref, v_ref):
    v_ref[...] = jnp.dot(x_ref[...], w_ref[...],
                         preferred_element_type=jnp.float32)


_proj = pl.pallas_call(
    _proj_body,
    grid=(_NBLK,),
    in_specs=[_row_spec(128), _full_spec(128, 64)],
    out_specs=_row_spec(64),
    out_shape=jax.ShapeDtypeStruct((N_NODES, 64), jnp.float32),
)


def _mid_body(eps_ref, v_ref, agg_ref, ba_ref, wb_ref, bb_ref,
              w2a_ref, emb_ref, u_ref, a0_v, a1_v, sem0, sem1):
    i = pl.program_id(0)
    n = N_NODES
    pltpu.make_async_copy(agg_ref.at[pl.ds(i * _BLOCK, _BLOCK)],
                          a0_v, sem0).start()
    pltpu.make_async_copy(agg_ref.at[pl.ds(n + i * _BLOCK, _BLOCK)],
                          a1_v, sem1).start()
    pltpu.make_async_copy(agg_ref.at[pl.ds(i * _BLOCK, _BLOCK)],
                          a0_v, sem0).wait()
    pltpu.make_async_copy(agg_ref.at[pl.ds(n + i * _BLOCK, _BLOCK)],
                          a1_v, sem1).wait()
    t = jnp.maximum((1.0 + eps_ref[0]) * v_ref[...] + a0_v[...]
                    + a1_v[...] + ba_ref[...], 0.0)
    emb = jnp.dot(t, wb_ref[...], preferred_element_type=jnp.float32) \
        + bb_ref[...]
    emb_ref[...] = emb
    h2 = jnp.maximum(emb, 0.0)
    u_ref[...] = jnp.dot(h2, w2a_ref[...], preferred_element_type=jnp.float32)


_mid = pl.pallas_call(
    _mid_body,
    grid=(_NBLK,),
    in_specs=[
        pl.BlockSpec(memory_space=pltpu.SMEM),
        _row_spec(64), pl.BlockSpec(memory_space=pltpu.HBM),
        _full_spec(1, 64), _full_spec(64, 64), _full_spec(1, 64),
        _full_spec(64, 64),
    ],
    out_specs=[_row_spec(64), _row_spec(64)],
    out_shape=[jax.ShapeDtypeStruct((N_NODES, 64), jnp.float32),
               jax.ShapeDtypeStruct((N_NODES, 64), jnp.float32)],
    scratch_shapes=[
        pltpu.VMEM((_BLOCK, 64), jnp.float32),
        pltpu.VMEM((_BLOCK, 64), jnp.float32),
        pltpu.SemaphoreType.DMA,
        pltpu.SemaphoreType.DMA,
    ],
)


def _out_body(eps_ref, u_ref, agg_ref, ba_ref, wb_ref, bb_ref,
              o_ref, a0_v, a1_v, sem0, sem1):
    i = pl.program_id(0)
    n = N_NODES
    pltpu.make_async_copy(agg_ref.at[pl.ds(i * _BLOCK, _BLOCK)],
                          a0_v, sem0).start()
    pltpu.make_async_copy(agg_ref.at[pl.ds(n + i * _BLOCK, _BLOCK)],
                          a1_v, sem1).start()
    pltpu.make_async_copy(agg_ref.at[pl.ds(i * _BLOCK, _BLOCK)],
                          a0_v, sem0).wait()
    pltpu.make_async_copy(agg_ref.at[pl.ds(n + i * _BLOCK, _BLOCK)],
                          a1_v, sem1).wait()
    t = jnp.maximum((1.0 + eps_ref[0]) * u_ref[...] + a0_v[...]
                    + a1_v[...] + ba_ref[...], 0.0)
    o_ref[...] = jnp.dot(t, wb_ref[...], preferred_element_type=jnp.float32) \
        + bb_ref[...]


_out = pl.pallas_call(
    _out_body,
    grid=(_NBLK,),
    in_specs=[
        pl.BlockSpec(memory_space=pltpu.SMEM),
        _row_spec(64), pl.BlockSpec(memory_space=pltpu.HBM),
        _full_spec(1, 64), _full_spec(64, 64), _full_spec(1, 64),
    ],
    out_specs=_row_spec(64),
    out_shape=jax.ShapeDtypeStruct((N_NODES, 64), jnp.float32),
    scratch_shapes=[
        pltpu.VMEM((_BLOCK, 64), jnp.float32),
        pltpu.VMEM((_BLOCK, 64), jnp.float32),
        pltpu.SemaphoreType.DMA,
        pltpu.SemaphoreType.DMA,
    ],
)


def kernel(x, W1a, b1a, W1b, b1b, eps1, W2a, b2a, W2b, b2b, eps2, edge_index):
    n = x.shape[0]
    rps = (n // _NSUB) // 8 * 8
    z64 = jnp.zeros((rps, 64), jnp.float32)

    v = _proj(x, W1a)
    aggv = _segsum64(v, edge_index, z64)
    eps1v = jnp.reshape(eps1, (1,))
    emb, u = _mid(eps1v, v, aggv,
                  jnp.reshape(b1a, (1, -1)), W1b,
                  jnp.reshape(b1b, (1, -1)), W2a)

    aggu = _segsum64(u, edge_index, z64)
    eps2v = jnp.reshape(eps2, (1,))
    logits = _out(eps2v, u, aggu,
                  jnp.reshape(b2a, (1, -1)), W2b,
                  jnp.reshape(b2b, (1, -1)))
    return (logits, emb)


# speedup vs baseline: 1.0647x; 1.0647x over previous
import functools

import jax
import jax.numpy as jnp
from jax import lax
from jax.experimental import pallas as pl
from jax.experimental.pallas import tpu as pltpu
from jax.experimental.pallas import tpu_sc as plsc

N_NODES = 10000
N_EDGES = 320000

_NCORES = 2
_NSUB = 16
_CHUNK = 80


def _make_segsum(n, e, d, nbuf, tc_tiling):
    nw = _NCORES * _NSUB
    epw = e // nw
    nch = epw // _CHUNK
    rps = (n // _NSUB) // 8 * 8
    tail = n - rps * _NSUB
    assert tail % 8 == 0

    mesh = plsc.VectorSubcoreMesh(core_axis_name="c", subcore_axis_name="s")

    @functools.partial(
        pl.kernel,
        out_type=jax.ShapeDtypeStruct((_NCORES * n, d), jnp.float32),
        mesh=mesh,
        compiler_params=pltpu.CompilerParams(use_tc_tiling_on_sc=tc_tiling),
        scratch_types=[
            pltpu.VMEM((epw,), jnp.int32),
            pltpu.VMEM((epw,), jnp.int32),
            pltpu.VMEM((nbuf, _CHUNK, d), jnp.float32),
            pltpu.VMEM_SHARED((n, d), jnp.float32),
            pltpu.SemaphoreType.DMA((nbuf,)),
        ],
    )
    def segsum(x_hbm, edges_hbm, zeros_hbm, out_hbm,
               srcbuf, dstbuf, rows, acc, sems):
        c = lax.axis_index("c")
        s = lax.axis_index("s")
        w = c * _NSUB + s
        pltpu.sync_copy(zeros_hbm.at[pl.ds(0, rps)],
                        acc.at[pl.ds(s * rps, rps)])

        @pl.when(s == 0)
        def _():
            pltpu.sync_copy(zeros_hbm.at[pl.ds(0, tail)],
                            acc.at[pl.ds(rps * _NSUB, tail)])

        base = pl.multiple_of(w * epw, 8)
        pltpu.sync_copy(edges_hbm.at[0, pl.ds(base, epw)], srcbuf)
        pltpu.sync_copy(edges_hbm.at[1, pl.ds(base, epw)], dstbuf)
        plsc.subcore_barrier()

        def src_idx(g):
            return srcbuf.at[pl.ds(pl.multiple_of(g * _CHUNK, 8), _CHUNK)]

        def dst_idx(g):
            return dstbuf.at[pl.ds(pl.multiple_of(g * _CHUNK, 8), _CHUNK)]

        for b in range(nbuf):
            pltpu.async_copy(x_hbm.at[src_idx(b)], rows.at[b], sems.at[b])

        @pl.loop(0, nch, step=nbuf)
        def _(g0):
            for b in range(nbuf):
                g = g0 + b

                @pl.when(g < nch)
                def _():
                    pltpu.make_async_copy(x_hbm.at[src_idx(g)], rows.at[b],
                                          sems.at[b]).wait()
                    pltpu.sync_copy(rows.at[b], acc.at[dst_idx(g)],
                                    add=True)
                    nxt = g + nbuf

                    @pl.when(nxt < nch)
                    def _():
                        pltpu.async_copy(x_hbm.at[src_idx(nxt)],
                                         rows.at[b], sems.at[b])

        plsc.subcore_barrier()
        pltpu.sync_copy(acc.at[pl.ds(s * rps, rps)],
                        out_hbm.at[pl.ds(c * n + s * rps, rps)])

        @pl.when(s == 0)
        def _():
            pltpu.sync_copy(acc.at[pl.ds(rps * _NSUB, tail)],
                            out_hbm.at[pl.ds(c * n + rps * _NSUB, tail)])

    return segsum


_segsum64 = _make_segsum(N_NODES, N_EDGES, 64, 10, False)

_BLOCK = 2000
_NBLK = N_NODES // _BLOCK


def _row_spec(d):
    return pl.BlockSpec((_BLOCK, d), lambda i: (i, 0))


def _half_spec(d, half):
    return pl.BlockSpec((_BLOCK, d), lambda i, h=half: (i + h * _NBLK, 0))


def _full_spec(r, c):
    return pl.BlockSpec((r, c), lambda i: (0, 0))


def _proj_body(x_ref, w_ref, v_ref):
    v_ref[...] = jnp.dot(x_ref[...], w_ref[...],
                         preferred_element_type=jnp.float32)


_proj = pl.pallas_call(
    _proj_body,
    grid=(_NBLK,),
    in_specs=[_row_spec(128), _full_spec(128, 64)],
    out_specs=_row_spec(64),
    out_shape=jax.ShapeDtypeStruct((N_NODES, 64), jnp.float32),
)


def _mid_body(eps_ref, v_ref, a0_ref, a1_ref, ba_ref, wb_ref, bb_ref,
              w2a_ref, emb_ref, u_ref):
    t = jnp.maximum((1.0 + eps_ref[0]) * v_ref[...] + a0_ref[...]
                    + a1_ref[...] + ba_ref[...], 0.0)
    emb = jnp.dot(t, wb_ref[...], preferred_element_type=jnp.float32) \
        + bb_ref[...]
    emb_ref[...] = emb
    h2 = jnp.maximum(emb, 0.0)
    u_ref[...] = jnp.dot(h2, w2a_ref[...], preferred_element_type=jnp.float32)


_mid = pl.pallas_call(
    _mid_body,
    grid=(_NBLK,),
    in_specs=[
        pl.BlockSpec(memory_space=pltpu.SMEM),
        _row_spec(64), _half_spec(64, 0), _half_spec(64, 1),
        _full_spec(1, 64), _full_spec(64, 64), _full_spec(1, 64),
        _full_spec(64, 64),
    ],
    out_specs=[_row_spec(64), _row_spec(64)],
    out_shape=[jax.ShapeDtypeStruct((N_NODES, 64), jnp.float32),
               jax.ShapeDtypeStruct((N_NODES, 64), jnp.float32)],
)


def _out_body(eps_ref, u_ref, a0_ref, a1_ref, ba_ref, wb_ref, bb_ref,
              o_ref):
    t = jnp.maximum((1.0 + eps_ref[0]) * u_ref[...] + a0_ref[...]
                    + a1_ref[...] + ba_ref[...], 0.0)
    o_ref[...] = jnp.dot(t, wb_ref[...], preferred_element_type=jnp.float32) \
        + bb_ref[...]


_out = pl.pallas_call(
    _out_body,
    grid=(_NBLK,),
    in_specs=[
        pl.BlockSpec(memory_space=pltpu.SMEM),
        _row_spec(64), _half_spec(64, 0), _half_spec(64, 1),
        _full_spec(1, 64), _full_spec(64, 64), _full_spec(1, 64),
    ],
    out_specs=_row_spec(64),
    out_shape=jax.ShapeDtypeStruct((N_NODES, 64), jnp.float32),
)


def kernel(x, W1a, b1a, W1b, b1b, eps1, W2a, b2a, W2b, b2b, eps2, edge_index):
    n = x.shape[0]
    rps = (n // _NSUB) // 8 * 8
    z64 = jnp.zeros((rps, 64), jnp.float32)

    v = _proj(x, W1a)
    aggv = _segsum64(v, edge_index, z64)
    eps1v = jnp.reshape(eps1, (1,))
    emb, u = _mid(eps1v, v, aggv, aggv,
                  jnp.reshape(b1a, (1, -1)), W1b,
                  jnp.reshape(b1b, (1, -1)), W2a)

    aggu = _segsum64(u, edge_index, z64)
    eps2v = jnp.reshape(eps2, (1,))
    logits = _out(eps2v, u, aggu, aggu,
                  jnp.reshape(b2a, (1, -1)), W2b,
                  jnp.reshape(b2b, (1, -1)))
    return (logits, emb)
